# baseline (speedup 1.0000x reference)
import jax
import jax.numpy as jnp
from jax.experimental import pallas as pl

_TILE = 256
_KSPLIT = 4
_ALPHA = 0.2


def _prep_kernel(x_ref, w_ref, a_ref, h_ref, s_ref, t_ref):
    f = w_ref.shape[1]
    h = jnp.dot(x_ref[...], w_ref[...], preferred_element_type=jnp.float32)
    h_ref[...] = h
    na_src = -a_ref[...][:, :f]
    na_dst = -a_ref[...][:, f:]
    s_ref[...] = jax.lax.dot_general(h, na_src, (((1,), (1,)), ((), ())),
                                     preferred_element_type=jnp.float32)
    t_ref[...] = jax.lax.dot_general(na_dst, h, (((1,), (1,)), ((), ())),
                                     preferred_element_type=jnp.float32)


def _main_kernel(h_ref, s_ref, t_ref, *rest):
    adj_refs = rest[:_KSPLIT]
    out_ref = rest[_KSPLIT]
    h = h_ref[...]
    s = s_ref[...]
    t = t_ref[...]
    n = h.shape[0]
    c = n // _KSPLIT
    acc = None
    rowsum = None
    for k in range(_KSPLIT):
        z = s + t[:, k * c:(k + 1) * c]
        e = jnp.exp(jnp.minimum(z, _ALPHA * z))
        e = jnp.where(adj_refs[k][...], e, 0.0)
        ones_col = jnp.ones((c, 1), dtype=jnp.float32)
        rs_k = jnp.dot(e, ones_col, preferred_element_type=jnp.float32)
        hp_k = jnp.dot(e, h[k * c:(k + 1) * c, :],
                       preferred_element_type=jnp.float32)
        acc = hp_k if acc is None else acc + hp_k
        rowsum = rs_k if rowsum is None else rowsum + rs_k
    hp = acc / rowsum
    out_ref[...] = jnp.where(hp > 0, hp, jnp.exp(hp) - 1.0)


def kernel(input, adj, W, a):
    n, in_f = input.shape
    out_f = W.shape[1]
    c = n // _KSPLIT

    h, s, t = pl.pallas_call(
        _prep_kernel,
        out_shape=[
            jax.ShapeDtypeStruct((n, out_f), jnp.float32),
            jax.ShapeDtypeStruct((n, 1), jnp.float32),
            jax.ShapeDtypeStruct((1, n), jnp.float32),
        ],
    )(input, W, a)

    grid = (n // _TILE,)
    in_specs = [
        pl.BlockSpec((n, out_f), lambda i: (0, 0)),
        pl.BlockSpec((_TILE, 1), lambda i: (i, 0)),
        pl.BlockSpec((1, n), lambda i: (0, 0)),
    ]
    for k in range(_KSPLIT):
        in_specs.append(pl.BlockSpec((_TILE, c), lambda i, k=k: (i, k)))

    return pl.pallas_call(
        _main_kernel,
        grid=grid,
        in_specs=in_specs,
        out_specs=pl.BlockSpec((_TILE, out_f), lambda i: (i, 0)),
        out_shape=jax.ShapeDtypeStruct((n, out_f), jnp.float32),
    )(h, s, t, *([adj] * _KSPLIT))


# speedup vs baseline: 1.3600x; 1.3600x over previous
import jax
import jax.numpy as jnp
from jax.experimental import pallas as pl

_TILE = 256
_ALPHA = 0.2


def _gat_tile_kernel(x_ref, x_tile_ref, adj_ref, w_ref, a_ref, out_ref):
    f = w_ref.shape[1]
    h_all = jnp.dot(x_ref[...], w_ref[...], preferred_element_type=jnp.float32)
    a_vec = a_ref[...]
    na_src = -a_vec[:, :f]
    na_dst = -a_vec[:, f:]

    h_i = jnp.dot(x_tile_ref[...], w_ref[...], preferred_element_type=jnp.float32)

    s = jax.lax.dot_general(h_i, na_src, (((1,), (1,)), ((), ())),
                            preferred_element_type=jnp.float32)
    t = jax.lax.dot_general(na_dst, h_all, (((1,), (1,)), ((), ())),
                            preferred_element_type=jnp.float32)

    z = s + t
    e = jnp.exp(jnp.minimum(z, _ALPHA * z))
    e = jnp.where(adj_ref[...] != 0, e, 0.0)

    ones_col = jnp.ones((h_all.shape[0], 1), dtype=jnp.float32)
    rowsum = jnp.dot(e, ones_col, preferred_element_type=jnp.float32)
    hp = jnp.dot(e, h_all, preferred_element_type=jnp.float32)
    hp = hp / rowsum
    out_ref[...] = jnp.where(hp > 0, hp, jnp.exp(hp) - 1.0)


def kernel(input, adj, W, a):
    n, in_f = input.shape
    out_f = W.shape[1]
    adj_i8 = adj.astype(jnp.int8)
    grid = (n // _TILE,)
    return pl.pallas_call(
        _gat_tile_kernel,
        grid=grid,
        in_specs=[
            pl.BlockSpec((n, in_f), lambda i: (0, 0)),
            pl.BlockSpec((_TILE, in_f), lambda i: (i, 0)),
            pl.BlockSpec((_TILE, n), lambda i: (i, 0)),
            pl.BlockSpec((in_f, out_f), lambda i: (0, 0)),
            pl.BlockSpec((1, 2 * out_f), lambda i: (0, 0)),
        ],
        out_specs=pl.BlockSpec((_TILE, out_f), lambda i: (i, 0)),
        out_shape=jax.ShapeDtypeStruct((n, out_f), jnp.float32),
    )(input, input, adj_i8, W, a)
